# baseline (speedup 1.0000x reference)
import functools

import jax
import jax.numpy as jnp
from jax import lax
from jax.experimental import pallas as pl
from jax.experimental.pallas import tpu as pltpu
from jax.experimental.pallas import tpu_sc as plsc

_K = 8192
_SHAPE = (4096, 4096)
_N = _SHAPE[0] * _SHAPE[1]

_NC = 2
_NS = 16
_NW = _NC * _NS
_LANES = 16

_BR = 8
_BC = 512
_BLK = _BR * _BC
_NBLK = _N // _BLK
_PER_W = _NBLK // _NW
_CPR = _SHAPE[1] // _BC
_NBUF = 8
_CSHIFT = (_BC // _LANES - 1).bit_length()
_CMASK = (1 << _CSHIFT) - 1


def _sc_body(cent_hbm, idx_hbm, mask_hbm, out_hbm, cent_ref, *rest):
    idx_bufs = rest[0:_NBUF]
    mask_bufs = rest[_NBUF:2 * _NBUF]
    out_bufs = rest[2 * _NBUF:3 * _NBUF]
    sem_in = rest[3 * _NBUF:4 * _NBUF]
    sem_out = rest[4 * _NBUF:5 * _NBUF]

    wid = lax.axis_index("s") * _NC + lax.axis_index("c")
    base_q = wid * _PER_W

    pltpu.sync_copy(cent_hbm, cent_ref)

    def block_slice(ref, q):
        rb = (q // _CPR) * _BR
        cb = (q % _CPR) * _BC
        return ref.at[pl.ds(rb, _BR), pl.ds(cb, _BC)]

    def start_in(q, b):
        pltpu.async_copy(block_slice(idx_hbm, q), idx_bufs[b], sem_in[b])
        pltpu.async_copy(block_slice(mask_hbm, q), mask_bufs[b], sem_in[b])

    for b in range(_NBUF):
        start_in(base_q + b, b)

    def outer(g, carry):
        for b in range(_NBUF):
            q = base_q + _NBUF * g + b
            ib, mb, ob = idx_bufs[b], mask_bufs[b], out_bufs[b]

            pltpu.make_async_copy(block_slice(idx_hbm, q), ib,
                                  sem_in[b]).wait()
            pltpu.make_async_copy(block_slice(mask_hbm, q), mb,
                                  sem_in[b]).wait()

            @pl.when(g > 0)
            def _():
                pltpu.make_async_copy(ob, block_slice(out_hbm, q - _NBUF),
                                      sem_out[b]).wait()

            @plsc.parallel_loop(0, _BLK // _LANES, 1, unroll=16)
            def _(qq):
                r = qq >> _CSHIFT
                off = (qq & _CMASK) * _LANES
                iv = ib[r, pl.ds(off, _LANES)]
                sv = jnp.maximum(iv, 0)
                g16 = plsc.load_gather(cent_ref, [sv])
                ob[r, pl.ds(off, _LANES)] = g16 * mb[r, pl.ds(off, _LANES)]

            pltpu.async_copy(ob, block_slice(out_hbm, q), sem_out[b])

            @pl.when(g < _PER_W // _NBUF - 1)
            def _():
                start_in(q + _NBUF, b)
        return carry

    lax.fori_loop(0, _PER_W // _NBUF, outer, 0)

    for b in range(_NBUF):
        last = base_q + _PER_W - _NBUF + b
        pltpu.make_async_copy(out_bufs[b], block_slice(out_hbm, last),
                              sem_out[b]).wait()


def kernel(cent, idx, mask):
    mesh = plsc.VectorSubcoreMesh(core_axis_name="c", subcore_axis_name="s")
    out = pl.kernel(
        _sc_body,
        mesh=mesh,
        compiler_params=pltpu.CompilerParams(
            needs_layout_passes=False,
            use_tc_tiling_on_sc=True,
        ),
        out_type=jax.ShapeDtypeStruct(_SHAPE, jnp.float32),
        scratch_types=[
            pltpu.VMEM((_K,), jnp.float32),
            *[pltpu.VMEM((_BR, _BC), jnp.int32) for _ in range(_NBUF)],
            *[pltpu.VMEM((_BR, _BC), jnp.float32) for _ in range(_NBUF)],
            *[pltpu.VMEM((_BR, _BC), jnp.float32) for _ in range(_NBUF)],
            *[pltpu.SemaphoreType.DMA for _ in range(2 * _NBUF)],
        ],
    )(cent, idx, mask)
    return out


# speedup vs baseline: 1.0253x; 1.0253x over previous
import functools

import jax
import jax.numpy as jnp
from jax import lax
from jax.experimental import pallas as pl
from jax.experimental.pallas import tpu as pltpu
from jax.experimental.pallas import tpu_sc as plsc

_K = 8192
_SHAPE = (4096, 4096)
_N = _SHAPE[0] * _SHAPE[1]

_NC = 2
_NS = 16
_NW = _NC * _NS
_LANES = 16

_BR = 8
_BC = 1024
_BLK = _BR * _BC
_NBLK = _N // _BLK
_PER_W = _NBLK // _NW
_CPR = _SHAPE[1] // _BC
_NBUF = 4
_CSHIFT = (_BC // _LANES - 1).bit_length()
_CMASK = (1 << _CSHIFT) - 1


def _sc_body(cent_hbm, idx_hbm, mask_hbm, out_hbm, cent_ref, *rest):
    idx_bufs = rest[0:_NBUF]
    mask_bufs = rest[_NBUF:2 * _NBUF]
    out_bufs = rest[2 * _NBUF:3 * _NBUF]
    sem_in = rest[3 * _NBUF:4 * _NBUF]
    sem_out = rest[4 * _NBUF:5 * _NBUF]

    wid = lax.axis_index("s") * _NC + lax.axis_index("c")
    base_q = wid * _PER_W

    pltpu.sync_copy(cent_hbm, cent_ref)

    def block_slice(ref, q):
        rb = (q // _CPR) * _BR
        cb = (q % _CPR) * _BC
        return ref.at[pl.ds(rb, _BR), pl.ds(cb, _BC)]

    def start_in(q, b):
        pltpu.async_copy(block_slice(idx_hbm, q), idx_bufs[b], sem_in[b])
        pltpu.async_copy(block_slice(mask_hbm, q), mask_bufs[b], sem_in[b])

    for b in range(_NBUF):
        start_in(base_q + b, b)

    def outer(g, carry):
        for b in range(_NBUF):
            q = base_q + _NBUF * g + b
            ib, mb, ob = idx_bufs[b], mask_bufs[b], out_bufs[b]

            pltpu.make_async_copy(block_slice(idx_hbm, q), ib,
                                  sem_in[b]).wait()
            pltpu.make_async_copy(block_slice(mask_hbm, q), mb,
                                  sem_in[b]).wait()

            @pl.when(g > 0)
            def _():
                pltpu.make_async_copy(ob, block_slice(out_hbm, q - _NBUF),
                                      sem_out[b]).wait()

            @plsc.parallel_loop(0, _BLK // _LANES, 1, unroll=16)
            def _(qq):
                r = qq >> _CSHIFT
                off = (qq & _CMASK) * _LANES
                iv = ib[r, pl.ds(off, _LANES)]
                sv = jnp.maximum(iv, 0)
                g16 = plsc.load_gather(cent_ref, [sv])
                ob[r, pl.ds(off, _LANES)] = g16 * mb[r, pl.ds(off, _LANES)]

            pltpu.async_copy(ob, block_slice(out_hbm, q), sem_out[b])

            @pl.when(g < _PER_W // _NBUF - 1)
            def _():
                start_in(q + _NBUF, b)
        return carry

    lax.fori_loop(0, _PER_W // _NBUF, outer, 0)

    for b in range(_NBUF):
        last = base_q + _PER_W - _NBUF + b
        pltpu.make_async_copy(out_bufs[b], block_slice(out_hbm, last),
                              sem_out[b]).wait()


def kernel(cent, idx, mask):
    mesh = plsc.VectorSubcoreMesh(core_axis_name="c", subcore_axis_name="s")
    out = pl.kernel(
        _sc_body,
        mesh=mesh,
        compiler_params=pltpu.CompilerParams(
            needs_layout_passes=False,
            use_tc_tiling_on_sc=True,
        ),
        out_type=jax.ShapeDtypeStruct(_SHAPE, jnp.float32),
        scratch_types=[
            pltpu.VMEM((_K,), jnp.float32),
            *[pltpu.VMEM((_BR, _BC), jnp.int32) for _ in range(_NBUF)],
            *[pltpu.VMEM((_BR, _BC), jnp.float32) for _ in range(_NBUF)],
            *[pltpu.VMEM((_BR, _BC), jnp.float32) for _ in range(_NBUF)],
            *[pltpu.SemaphoreType.DMA for _ in range(2 * _NBUF)],
        ],
    )(cent, idx, mask)
    return out
